# Initial kernel scaffold; baseline (speedup 1.0000x reference)
import functools

import jax
import jax.numpy as jnp
from jax import lax
from jax.experimental import pallas as pl
from jax.experimental.pallas import tpu as pltpu
from jax.experimental.pallas import tpu_sc as plsc

NUM_TYPES = 21
EMB = 128
NUM_ATOMS = 524288
NC, NS = 2, 16
NW = NC * NS
CHUNK = 128
ROWS_PER_W = NUM_ATOMS // NW
CHUNKS_PER_W = ROWS_PER_W // CHUNK
NBUF = 2


def _make_gather():
  mesh = plsc.VectorSubcoreMesh(core_axis_name="c", subcore_axis_name="s")

  @functools.partial(
      pl.kernel,
      out_type=jax.ShapeDtypeStruct((NUM_ATOMS, EMB), jnp.float32),
      mesh=mesh,
      scratch_types=[
          pltpu.VMEM_SHARED((NUM_TYPES, EMB), jnp.float32),
          pltpu.VMEM((CHUNKS_PER_W, CHUNK), jnp.int32),
          pltpu.VMEM((CHUNK, EMB), jnp.float32),
          pltpu.VMEM((CHUNK, EMB), jnp.float32),
          pltpu.SemaphoreType.DMA,
          pltpu.SemaphoreType.DMA,
      ],
  )
  def k(w_hbm, idx_hbm, out_hbm, w_sh, idx_v, rows0, rows1, sem0, sem1):
    cid = lax.axis_index("c")
    sid = lax.axis_index("s")
    wid = sid * NC + cid

    @pl.when(sid == 0)
    def _():
      pltpu.sync_copy(w_hbm, w_sh)

    plsc.subcore_barrier()

    base_row = wid * CHUNKS_PER_W
    pltpu.sync_copy(idx_hbm.at[pl.ds(base_row, CHUNKS_PER_W)], idx_v)

    rows = (rows0, rows1)
    sems = (sem0, sem1)

    def start(j, b):
      pltpu.async_copy(w_sh.at[idx_v.at[j]], rows[b], sems[b])

    def wait(b):
      pltpu.make_async_copy(w_sh.at[idx_v.at[0]], rows[b], sems[b]).wait()

    start(0, 0)
    start(1, 1)

    def body(j2, carry):
      for b in range(NBUF):
        j = j2 * NBUF + b
        wait(b)
        pltpu.sync_copy(rows[b],
                        out_hbm.at[pl.ds((base_row + j) * CHUNK, CHUNK)])

        @pl.when(j + NBUF < CHUNKS_PER_W)
        def _():
          start(j + NBUF, b)

      return carry

    lax.fori_loop(0, CHUNKS_PER_W // NBUF, body, 0)

  return k


_gather = _make_gather()


def kernel(residue_indices, W):
  idx = residue_indices.astype(jnp.int32).reshape(NUM_ATOMS // CHUNK, CHUNK)
  return _gather(W, idx)


# speedup vs baseline: 15.8979x; 15.8979x over previous
import functools

import jax
import jax.numpy as jnp
from jax import lax
from jax.experimental import pallas as pl
from jax.experimental.pallas import tpu as pltpu
from jax.experimental.pallas import tpu_sc as plsc

NUM_TYPES = 21
EMB = 128
NUM_ATOMS = 524288
NC, NS = 2, 16
NW = NC * NS
CHUNK = 128
ROWS_PER_W = NUM_ATOMS // NW
CHUNKS_PER_W = ROWS_PER_W // CHUNK
NBUF = 2


def _make_gather():
  mesh = plsc.VectorSubcoreMesh(core_axis_name="c", subcore_axis_name="s")

  @functools.partial(
      pl.kernel,
      out_type=jax.ShapeDtypeStruct((NUM_ATOMS, EMB), jnp.float32),
      mesh=mesh,
      scratch_types=[
          pltpu.VMEM_SHARED((NUM_TYPES, EMB), jnp.float32),
          pltpu.VMEM((CHUNKS_PER_W, CHUNK), jnp.int32),
          pltpu.VMEM((CHUNK, EMB), jnp.float32),
          pltpu.VMEM((CHUNK, EMB), jnp.float32),
          pltpu.SemaphoreType.DMA,
          pltpu.SemaphoreType.DMA,
      ],
  )
  def k(w_hbm, idx_hbm, out_hbm, w_sh, idx_v, rows0, rows1, sem0, sem1):
    cid = lax.axis_index("c")
    sid = lax.axis_index("s")
    wid = sid * NC + cid

    @pl.when(sid == 0)
    def _():
      w_stage = rows0.at[pl.ds(0, NUM_TYPES)]
      pltpu.sync_copy(w_hbm, w_stage)
      pltpu.sync_copy(w_stage, w_sh)

    plsc.subcore_barrier()

    base_row = wid * CHUNKS_PER_W
    pltpu.sync_copy(idx_hbm.at[pl.ds(base_row, CHUNKS_PER_W)], idx_v)

    rows = (rows0, rows1)
    sems = (sem0, sem1)

    def start(j, b):
      pltpu.async_copy(w_sh.at[idx_v.at[j]], rows[b], sems[b])

    def wait(b):
      pltpu.make_async_copy(w_sh.at[idx_v.at[0]], rows[b], sems[b]).wait()

    start(0, 0)
    start(1, 1)

    def body(j2, carry):
      for b in range(NBUF):
        j = j2 * NBUF + b
        wait(b)
        pltpu.sync_copy(rows[b],
                        out_hbm.at[pl.ds((base_row + j) * CHUNK, CHUNK)])

        @pl.when(j + NBUF < CHUNKS_PER_W)
        def _():
          start(j + NBUF, b)

      return carry

    lax.fori_loop(0, CHUNKS_PER_W // NBUF, body, 0)

  return k


_gather = _make_gather()


def kernel(residue_indices, W):
  idx = residue_indices.astype(jnp.int32).reshape(NUM_ATOMS // CHUNK, CHUNK)
  return _gather(W, idx)


# speedup vs baseline: 16.1827x; 1.0179x over previous
import functools

import jax
import jax.numpy as jnp
from jax import lax
from jax.experimental import pallas as pl
from jax.experimental.pallas import tpu as pltpu
from jax.experimental.pallas import tpu_sc as plsc

NUM_TYPES = 21
EMB = 128
NUM_ATOMS = 524288
NC, NS = 2, 16
NW = NC * NS
CHUNK = 128
ROWS_PER_W = NUM_ATOMS // NW
CHUNKS_PER_W = ROWS_PER_W // CHUNK
NBUF = 4


def _make_gather():
  mesh = plsc.VectorSubcoreMesh(core_axis_name="c", subcore_axis_name="s")

  @functools.partial(
      pl.kernel,
      out_type=jax.ShapeDtypeStruct((NUM_ATOMS, EMB), jnp.float32),
      mesh=mesh,
      scratch_types=[
          pltpu.VMEM_SHARED((NUM_TYPES, EMB), jnp.float32),
          pltpu.VMEM((CHUNKS_PER_W, CHUNK), jnp.int32),
          pltpu.VMEM((NBUF, CHUNK, EMB), jnp.float32),
          pltpu.SemaphoreType.DMA,
          [pltpu.SemaphoreType.DMA] * NBUF,
          [pltpu.SemaphoreType.DMA] * NBUF,
      ],
  )
  def k(w_hbm, idx_hbm, out_hbm, w_sh, idx_v, rows, isem, gsems, wsems):
    cid = lax.axis_index("c")
    sid = lax.axis_index("s")
    wid = sid * NC + cid
    base_row = wid * CHUNKS_PER_W

    idx_copy = pltpu.make_async_copy(
        idx_hbm.at[pl.ds(base_row, CHUNKS_PER_W)], idx_v, isem)
    idx_copy.start()

    @pl.when(sid == 0)
    def _():
      w_stage = rows.at[0].at[pl.ds(0, NUM_TYPES)]
      pltpu.sync_copy(w_hbm, w_stage)
      pltpu.sync_copy(w_stage, w_sh)

    plsc.subcore_barrier()
    idx_copy.wait()

    def start_gather(j, b):
      pltpu.async_copy(w_sh.at[idx_v.at[j]], rows.at[b], gsems[b])

    def wait_gather(b):
      pltpu.make_async_copy(w_sh.at[idx_v.at[0]], rows.at[b], gsems[b]).wait()

    def start_write(j, b):
      pltpu.make_async_copy(
          rows.at[b], out_hbm.at[pl.ds((base_row + j) * CHUNK, CHUNK)],
          wsems[b]).start()

    def wait_write(b):
      pltpu.make_async_copy(
          rows.at[b], out_hbm.at[pl.ds(base_row * CHUNK, CHUNK)],
          wsems[b]).wait()

    start_gather(0, 0)
    start_gather(1, 1)

    def body(j4, carry):
      for b in range(NBUF):
        j = j4 * NBUF + b
        wait_gather(b)
        start_write(j, b)
        jn = j + 2
        bn = (b + 2) % NBUF

        @pl.when(jnp.logical_and(jn >= NBUF, jn < CHUNKS_PER_W))
        def _():
          wait_write(bn)

        @pl.when(jn < CHUNKS_PER_W)
        def _():
          start_gather(jn, bn)

      return carry

    lax.fori_loop(0, CHUNKS_PER_W // NBUF, body, 0)

    for b in range(NBUF):
      wait_write(b)

  return k


_gather = _make_gather()


def kernel(residue_indices, W):
  idx = residue_indices.astype(jnp.int32).reshape(NUM_ATOMS // CHUNK, CHUNK)
  return _gather(W, idx)


# speedup vs baseline: 16.2110x; 1.0017x over previous
import functools

import jax
import jax.numpy as jnp
from jax import lax
from jax.experimental import pallas as pl
from jax.experimental.pallas import tpu as pltpu
from jax.experimental.pallas import tpu_sc as plsc

NUM_TYPES = 21
EMB = 128
NUM_ATOMS = 524288
NC, NS = 2, 16
NW = NC * NS
CHUNK = 128
ROWS_PER_W = NUM_ATOMS // NW
CHUNKS_PER_W = ROWS_PER_W // CHUNK
NBUF = 4


def _make_gather():
  mesh = plsc.VectorSubcoreMesh(core_axis_name="c", subcore_axis_name="s")

  @functools.partial(
      pl.kernel,
      out_type=jax.ShapeDtypeStruct((NUM_ATOMS, EMB), jnp.float32),
      mesh=mesh,
      scratch_types=[
          pltpu.VMEM_SHARED((NS * NUM_TYPES, EMB), jnp.float32),
          pltpu.VMEM((CHUNKS_PER_W, CHUNK), jnp.int32),
          pltpu.VMEM((NBUF, CHUNK, EMB), jnp.float32),
          pltpu.SemaphoreType.DMA,
          [pltpu.SemaphoreType.DMA] * NBUF,
          [pltpu.SemaphoreType.DMA] * NBUF,
      ],
  )
  def k(w_hbm, idx_hbm, out_hbm, w_sh, idx_v, rows, isem, gsems, wsems):
    cid = lax.axis_index("c")
    sid = lax.axis_index("s")
    wid = sid * NC + cid
    base_row = wid * CHUNKS_PER_W

    idx_copy = pltpu.make_async_copy(
        idx_hbm.at[pl.ds(base_row, CHUNKS_PER_W)], idx_v, isem)
    idx_copy.start()

    w_my = w_sh.at[pl.ds(sid * NUM_TYPES, NUM_TYPES)]
    w_stage = rows.at[0].at[pl.ds(0, NUM_TYPES)]
    pltpu.sync_copy(w_hbm, w_stage)
    pltpu.sync_copy(w_stage, w_my)
    idx_copy.wait()

    def start_gather(j, b):
      pltpu.async_copy(w_my.at[idx_v.at[j]], rows.at[b], gsems[b])

    def wait_gather(b):
      pltpu.make_async_copy(w_my.at[idx_v.at[0]], rows.at[b], gsems[b]).wait()

    def start_write(j, b):
      pltpu.make_async_copy(
          rows.at[b], out_hbm.at[pl.ds((base_row + j) * CHUNK, CHUNK)],
          wsems[b]).start()

    def wait_write(b):
      pltpu.make_async_copy(
          rows.at[b], out_hbm.at[pl.ds(base_row * CHUNK, CHUNK)],
          wsems[b]).wait()

    start_gather(0, 0)
    start_gather(1, 1)

    def body(j4, carry):
      for b in range(NBUF):
        j = j4 * NBUF + b
        wait_gather(b)
        start_write(j, b)
        jn = j + 2
        bn = (b + 2) % NBUF

        @pl.when(jnp.logical_and(jn >= NBUF, jn < CHUNKS_PER_W))
        def _():
          wait_write(bn)

        @pl.when(jn < CHUNKS_PER_W)
        def _():
          start_gather(jn, bn)

      return carry

    lax.fori_loop(0, CHUNKS_PER_W // NBUF, body, 0)

    for b in range(NBUF):
      wait_write(b)

  return k


_gather = _make_gather()


def kernel(residue_indices, W):
  idx = residue_indices.astype(jnp.int32).reshape(NUM_ATOMS // CHUNK, CHUNK)
  return _gather(W, idx)
